# baseline (speedup 1.0000x reference)
import functools

import jax
import jax.numpy as jnp
import numpy as np
from jax import lax
from jax.experimental import pallas as pl
from jax.experimental.pallas import tpu as pltpu
from jax.experimental.pallas import tpu_sc as plsc

_B = 64
_SPATIAL = 1024
_CDIM = 768
_SS = 32
_PS = 16
_NPOOL = 256
_TDIM = 32
_NC = 2
_NS = 16
_NW = _NC * _NS
_B_PER_W = _B // _NW


def _gather_indices() -> np.ndarray:
    ch = np.linspace(0, _CDIM - 1, _TDIM).astype(np.int64)
    out = np.empty((_B, _NPOOL, 4 * _TDIM), np.int32)
    for b in range(_B):
        for R in range(_PS):
            for C in range(_PS):
                p = R * _PS + C
                col = 0
                for dr in range(2):
                    for dc in range(2):
                        s = (2 * R + dr) * _SS + (2 * C + dc)
                        base = (b * _SPATIAL + s) * _CDIM
                        out[b, p, col * _TDIM:(col + 1) * _TDIM] = base + ch
                        col += 1
    return out.reshape(_B, _NPOOL * 4 * _TDIM)


def _sc_body(feat_hbm, idx_hbm, out_hbm, idx_v, data_v, out_v, sem):
    wid = lax.axis_index("s") * _NC + lax.axis_index("c")

    for b_local in range(_B_PER_W):
        b = wid * _B_PER_W + b_local
        pltpu.sync_copy(idx_hbm.at[b], idx_v)
        pltpu.async_copy(feat_hbm.at[idx_v], data_v, sem).wait()

        def row_body(i, _):
            base = i * (4 * _TDIM)
            for h in range(2):
                acc = data_v[pl.ds(base + h * 16, 16)]
                for sp in range(1, 4):
                    acc = acc + data_v[pl.ds(base + sp * _TDIM + h * 16, 16)]
                out_v[i, pl.ds(h * 16, 16)] = acc * jnp.float32(0.25)
            return _

        lax.fori_loop(0, _NPOOL, row_body, None)
        pltpu.sync_copy(out_v, out_hbm.at[b])


def kernel(features):
    b, spatial, c = features.shape
    flat = features.reshape(b * spatial * c)
    idx = jnp.asarray(_gather_indices())

    sc_call = functools.partial(
        pl.kernel,
        mesh=plsc.VectorSubcoreMesh(core_axis_name="c", subcore_axis_name="s"),
        out_type=jax.ShapeDtypeStruct((_B, _NPOOL, _TDIM), jnp.float32),
        scratch_types=[
            pltpu.VMEM((_NPOOL * 4 * _TDIM,), jnp.int32),
            pltpu.VMEM((_NPOOL * 4 * _TDIM,), jnp.float32),
            pltpu.VMEM((_NPOOL, _TDIM), jnp.float32),
            pltpu.SemaphoreType.DMA,
        ],
    )(_sc_body)
    return sc_call(flat, idx)
